# baseline (speedup 1.0000x reference)
import functools

import jax
import jax.numpy as jnp
from jax.experimental import pallas as pl
from jax.experimental.pallas import tpu as pltpu

_BK = 2048
_BIGF = 3e7


def _nn_body(q_ref, k_ref, top_sim_ref, top_idx_ref, avg_ref,
             qn_ref, idxf_ref, ksum_ref, colmat_ref, *, nblk, bk, k_total):
    i = pl.program_id(0)

    @pl.when(i == 0)
    def _init():
        q = q_ref[...]
        qn_ref[...] = q / (jnp.sqrt(jnp.sum(q * q, axis=1, keepdims=True)) + 1e-8)
        top_sim_ref[...] = jnp.full(top_sim_ref.shape, -jnp.inf, jnp.float32)
        idxf_ref[...] = jnp.zeros(idxf_ref.shape, jnp.float32)
        ksum_ref[...] = jnp.zeros(ksum_ref.shape, jnp.float32)
        rowid = jax.lax.broadcasted_iota(jnp.int32, colmat_ref.shape, 0)
        colid = jax.lax.broadcasted_iota(jnp.int32, colmat_ref.shape, 1)
        cm = jnp.where(colid == 0, rowid // 256,
                       jnp.where(colid == 1, rowid % 256,
                                 jnp.where(colid == 2, 1, 0)))
        colmat_ref[...] = cm.astype(jnp.bfloat16)

    kb = k_ref[...]
    kn = kb / (jnp.sqrt(jnp.sum(kb * kb, axis=1, keepdims=True)) + 1e-8)
    sim = jax.lax.dot_general(
        qn_ref[...], kn, (((1,), (1,)), ((), ())),
        preferred_element_type=jnp.float32)

    ksum_ref[...] += jnp.sum(kn, axis=0, keepdims=True)

    def _select(s):
        local_max = jnp.max(s, axis=1, keepdims=True)
        match = (s == local_max).astype(jnp.bfloat16)
        r = jax.lax.dot_general(
            match, colmat_ref[...], (((1,), (0,)), ((), ())),
            preferred_element_type=jnp.float32)
        argf = r[:, 0:1] * 256.0 + r[:, 1:2]
        cnt = r[:, 2:3]
        better = local_max > top_sim_ref[...]

        @pl.when(jnp.max(cnt) < 1.5)
        def _unique():
            idxf_ref[...] = jnp.where(
                better, argf + jnp.float32(i * bk), idxf_ref[...])

        @pl.when(jnp.max(cnt) >= 1.5)
        def _tie_exact():
            colf = jax.lax.broadcasted_iota(
                jnp.int32, s.shape, 1).astype(jnp.float32)
            cand = jnp.where(s == local_max, colf, _BIGF)
            first = jnp.min(cand, axis=1, keepdims=True)
            idxf_ref[...] = jnp.where(
                better, first + jnp.float32(i * bk), idxf_ref[...])

        top_sim_ref[...] = jnp.where(better, local_max, top_sim_ref[...])

    tail_valid = k_total - (nblk - 1) * bk

    @pl.when(i < nblk - 1)
    def _full_block():
        _select(sim)

    @pl.when(i == nblk - 1)
    def _tail_block():
        col = jax.lax.broadcasted_iota(jnp.int32, sim.shape, 1)
        _select(jnp.where(col < tail_valid, sim, -jnp.inf))
        avg = jax.lax.dot_general(
            qn_ref[...], ksum_ref[...], (((1,), (1,)), ((), ())),
            preferred_element_type=jnp.float32)
        avg_ref[...] = avg * (1.0 / k_total)
        top_idx_ref[...] = idxf_ref[...].astype(jnp.int32)


def kernel(queries, keys, db_classes):
    q, d = queries.shape
    k = keys.shape[0]
    k_pad = ((k + _BK - 1) // _BK) * _BK
    nblk = k_pad // _BK
    if k_pad != k:
        keys = jnp.pad(keys, ((0, k_pad - k), (0, 0)))

    top_sim, top_idx, avg_sim = pl.pallas_call(
        functools.partial(_nn_body, nblk=nblk, bk=_BK, k_total=k),
        grid=(nblk,),
        in_specs=[
            pl.BlockSpec((q, d), lambda i: (0, 0)),
            pl.BlockSpec((_BK, d), lambda i: (i, 0)),
        ],
        out_specs=[
            pl.BlockSpec((q, 1), lambda i: (0, 0)),
            pl.BlockSpec((q, 1), lambda i: (0, 0)),
            pl.BlockSpec((q, 1), lambda i: (0, 0)),
        ],
        out_shape=[
            jax.ShapeDtypeStruct((q, 1), jnp.float32),
            jax.ShapeDtypeStruct((q, 1), jnp.int32),
            jax.ShapeDtypeStruct((q, 1), jnp.float32),
        ],
        scratch_shapes=[
            pltpu.VMEM((q, d), jnp.float32),
            pltpu.VMEM((q, 1), jnp.float32),
            pltpu.VMEM((1, d), jnp.float32),
            pltpu.VMEM((_BK, 8), jnp.bfloat16),
        ],
        compiler_params=pltpu.CompilerParams(
            dimension_semantics=("arbitrary",)),
    )(queries, keys)

    top_cls = jnp.take(db_classes, top_idx[:, 0], axis=0)
    return (top_sim, top_cls, avg_sim[:, 0])


# speedup vs baseline: 1.1829x; 1.1829x over previous
import functools

import jax
import jax.numpy as jnp
from jax.experimental import pallas as pl
from jax.experimental.pallas import tpu as pltpu

_BK = 4096
_BIGF = 3e7


def _nn_body(q_ref, k_ref, top_sim_ref, top_idx_ref, avg_ref,
             qn_ref, idxf_ref, ksum_ref, *, nblk, bk, k_total):
    i = pl.program_id(0)

    @pl.when(i == 0)
    def _init():
        q = q_ref[...]
        qn_ref[...] = q / (jnp.sqrt(jnp.sum(q * q, axis=1, keepdims=True)) + 1e-8)
        top_sim_ref[...] = jnp.full(top_sim_ref.shape, -jnp.inf, jnp.float32)
        idxf_ref[...] = jnp.zeros(idxf_ref.shape, jnp.float32)
        ksum_ref[...] = jnp.zeros(ksum_ref.shape, jnp.float32)

    kb = k_ref[...]
    kn = kb / (jnp.sqrt(jnp.sum(kb * kb, axis=1, keepdims=True)) + 1e-8)
    sim = jax.lax.dot_general(
        qn_ref[...], kn, (((1,), (1,)), ((), ())),
        preferred_element_type=jnp.float32)

    ksum_ref[...] += jnp.sum(kn, axis=0, keepdims=True)

    def _select(s):
        local_max = jnp.max(s, axis=1, keepdims=True)
        colf = jax.lax.broadcasted_iota(
            jnp.int32, s.shape, 1).astype(jnp.float32)
        cand = jnp.where(s == local_max, colf, _BIGF)
        local_argf = jnp.min(cand, axis=1, keepdims=True)
        better = local_max > top_sim_ref[...]
        idxf_ref[...] = jnp.where(
            better, local_argf + jnp.float32(i * bk), idxf_ref[...])
        top_sim_ref[...] = jnp.where(better, local_max, top_sim_ref[...])

    tail_valid = k_total - (nblk - 1) * bk

    @pl.when(i < nblk - 1)
    def _full_block():
        _select(sim)

    @pl.when(i == nblk - 1)
    def _tail_block():
        col = jax.lax.broadcasted_iota(jnp.int32, sim.shape, 1)
        _select(jnp.where(col < tail_valid, sim, -jnp.inf))
        avg = jax.lax.dot_general(
            qn_ref[...], ksum_ref[...], (((1,), (1,)), ((), ())),
            preferred_element_type=jnp.float32)
        avg_ref[...] = avg * (1.0 / k_total)
        top_idx_ref[...] = idxf_ref[...].astype(jnp.int32)


def kernel(queries, keys, db_classes):
    q, d = queries.shape
    k = keys.shape[0]
    k_pad = ((k + _BK - 1) // _BK) * _BK
    nblk = k_pad // _BK
    if k_pad != k:
        keys = jnp.pad(keys, ((0, k_pad - k), (0, 0)))

    top_sim, top_idx, avg_sim = pl.pallas_call(
        functools.partial(_nn_body, nblk=nblk, bk=_BK, k_total=k),
        grid=(nblk,),
        in_specs=[
            pl.BlockSpec((q, d), lambda i: (0, 0)),
            pl.BlockSpec((_BK, d), lambda i: (i, 0)),
        ],
        out_specs=[
            pl.BlockSpec((q, 1), lambda i: (0, 0)),
            pl.BlockSpec((q, 1), lambda i: (0, 0)),
            pl.BlockSpec((q, 1), lambda i: (0, 0)),
        ],
        out_shape=[
            jax.ShapeDtypeStruct((q, 1), jnp.float32),
            jax.ShapeDtypeStruct((q, 1), jnp.int32),
            jax.ShapeDtypeStruct((q, 1), jnp.float32),
        ],
        scratch_shapes=[
            pltpu.VMEM((q, d), jnp.float32),
            pltpu.VMEM((q, 1), jnp.float32),
            pltpu.VMEM((1, d), jnp.float32),
        ],
        compiler_params=pltpu.CompilerParams(
            dimension_semantics=("arbitrary",)),
    )(queries, keys)

    top_cls = jnp.take(db_classes, top_idx[:, 0], axis=0)
    return (top_sim, top_cls, avg_sim[:, 0])
